# baseline (speedup 1.0000x reference)
import dataclasses
import functools

import jax
import jax.numpy as jnp
from jax import lax
from jax.experimental import pallas as pl
from jax.experimental.pallas import tpu as pltpu
from jax.experimental.pallas import tpu_sc as plsc

_EMB = 128
_LANES = 16
_VPR = _EMB // _LANES

_NC = 2
_NS = 16
_NW = _NC * _NS

_UNROLL = 5
_LPAD = 128


def _pool_body(nb, nchunk, L, text_hbm, table_hbm, out_hbm, idx_v, rows0,
               rows1, stage_v, sem0, sem1):
    c = lax.axis_index("c")
    s = lax.axis_index("s")
    wid = s * _NC + c
    bpw = nb * nchunk
    base_b = wid * bpw
    pltpu.sync_copy(text_hbm.at[pl.ds(base_b * _LPAD, bpw * _LPAD)], idx_v)

    bufs = (rows0, rows1)
    sems = (sem0, sem1)

    def fire(ci, k):
        for b in range(nb):
            pltpu.async_copy(
                table_hbm.at[idx_v.at[pl.ds((ci * nb + b) * _LPAD, L)]],
                bufs[k].at[pl.ds(b * L, L)], sems[k])

    def drain(ci, k):
        for b in range(nb):
            pltpu.make_async_copy(
                table_hbm.at[idx_v.at[pl.ds((ci * nb + b) * _LPAD, L)]],
                bufs[k].at[pl.ds(b * L, L)], sems[k]).wait()

    def consume(ci, k):
        b0 = base_b + ci * nb

        @pl.loop(0, nb)
        def _acc(b):
            def lbody(t, accs, k=k):
                out = list(accs)
                for u in range(_UNROLL):
                    r = b * L + t * _UNROLL + u
                    for j in range(_VPR):
                        out[j] = out[j] + bufs[k][r, pl.ds(j * _LANES, _LANES)]
                return tuple(out)
            accs = lax.fori_loop(
                0, L // _UNROLL, lbody,
                tuple(jnp.zeros((_LANES,), jnp.float32) for _ in range(_VPR)))
            boff = (ci * nb + b) * _LPAD
            cnt = plsc.all_reduce_population_count(
                idx_v[pl.ds(boff, _LANES)] != 0)
            for t in range(1, _LPAD // _LANES):
                cnt = cnt + plsc.all_reduce_population_count(
                    idx_v[pl.ds(boff + t * _LANES, _LANES)] != 0)
            inv = 1.0 / jnp.maximum(cnt.astype(jnp.float32), 1.0)
            for j in range(_VPR):
                stage_v[b, pl.ds(j * _LANES, _LANES)] = accs[j] * inv

        pltpu.sync_copy(stage_v, out_hbm.at[pl.ds(b0, nb)])

    fire(0, 0)

    @pl.loop(0, nchunk, step=2)
    def _pair(ci):
        fire(ci + 1, 1)
        drain(ci, 0)
        consume(ci, 0)

        @pl.when(ci + 2 < nchunk)
        def _():
            fire(ci + 2, 0)

        drain(ci + 1, 1)
        consume(ci + 1, 1)


def _make_pool(B, L):
    nb = 8
    bpw = B // _NW
    nchunk = bpw // nb
    rows = nb * L
    mesh = plsc.VectorSubcoreMesh(core_axis_name="c", subcore_axis_name="s")
    cp = pltpu.CompilerParams()
    if "needs_layout_passes" in pltpu.CompilerParams.__dataclass_fields__:
        cp = dataclasses.replace(cp, needs_layout_passes=False)
    return pl.kernel(
        functools.partial(_pool_body, nb, nchunk, L),
        out_type=jax.ShapeDtypeStruct((B, _EMB), jnp.float32),
        mesh=mesh,
        compiler_params=cp,
        scratch_types=[
            pltpu.VMEM((bpw * _LPAD,), jnp.int32),
            pltpu.VMEM((rows, _EMB), jnp.float32),
            pltpu.VMEM((rows, _EMB), jnp.float32),
            pltpu.VMEM((nb, _EMB), jnp.float32),
            pltpu.SemaphoreType.DMA,
            pltpu.SemaphoreType.DMA,
        ],
    )


def _mlp_body(p_ref, W1_ref, b1_ref, W2_ref, b2_ref, W3_ref, b3_ref, o_ref):
    dn = (((1,), (1,)), ((), ()))
    h = lax.dot_general(p_ref[...].astype(jnp.bfloat16), W1_ref[...], dn,
                        preferred_element_type=jnp.float32) + b1_ref[...]
    h = jnp.maximum(h, 0.0)
    h = lax.dot_general(h.astype(jnp.bfloat16), W2_ref[...], dn,
                        preferred_element_type=jnp.float32) + b2_ref[...]
    h = jnp.maximum(h, 0.0)
    o_ref[...] = lax.dot_general(h.astype(jnp.bfloat16), W3_ref[...], dn,
                                 preferred_element_type=jnp.float32) + b3_ref[...]


def _mlp(pooled, W1, b1, W2, b2, W3, b3):
    B = pooled.shape[0]
    H1 = W1.shape[0]
    H2 = W2.shape[0]
    OUT = W3.shape[0]
    BM = 4096
    grid = (B // BM,)
    full = lambda i: (0, 0)
    return pl.pallas_call(
        _mlp_body,
        grid=grid,
        in_specs=[
            pl.BlockSpec((BM, _EMB), lambda i: (i, 0)),
            pl.BlockSpec((H1, _EMB), full),
            pl.BlockSpec((1, H1), full),
            pl.BlockSpec((H2, H1), full),
            pl.BlockSpec((1, H2), full),
            pl.BlockSpec((OUT, H2), full),
            pl.BlockSpec((1, OUT), full),
        ],
        out_specs=pl.BlockSpec((BM, OUT), lambda i: (i, 0)),
        out_shape=jax.ShapeDtypeStruct((B, OUT), jnp.float32),
    )(pooled, W1, b1, W2, b2, W3, b3)


def kernel(text, table, W1, b1, W2, b2, W3, b3):
    B, L = text.shape
    pool = _make_pool(B, L)
    W1b = W1.astype(jnp.bfloat16)
    W2b = W2.astype(jnp.bfloat16)
    W3b = W3.astype(jnp.bfloat16)
    tp = jnp.pad(text.astype(jnp.int32), ((0, 0), (0, _LPAD - L)))
    pooled = pool(tp.reshape(-1), table)
    return _mlp(pooled, W1b, b1.reshape(1, -1), W2b, b2.reshape(1, -1),
                W3b, b3.reshape(1, -1))


# speedup vs baseline: 1.0002x; 1.0002x over previous
import dataclasses
import functools

import jax
import jax.numpy as jnp
from jax import lax
from jax.experimental import pallas as pl
from jax.experimental.pallas import tpu as pltpu
from jax.experimental.pallas import tpu_sc as plsc

_EMB = 128
_LANES = 16
_VPR = _EMB // _LANES

_NC = 2
_NS = 16
_NW = _NC * _NS

_UNROLL = 5
_LPAD = 128


def _pool_body(nb, nchunk, L, text_hbm, table_hbm, out_hbm, idx_v, rows0,
               rows1, stage_v, sem0, sem1):
    c = lax.axis_index("c")
    s = lax.axis_index("s")
    wid = s * _NC + c
    bpw = nb * nchunk
    base_b = wid * bpw
    pltpu.sync_copy(text_hbm.at[pl.ds(base_b * _LPAD, bpw * _LPAD)], idx_v)

    bufs = (rows0, rows1)
    sems = (sem0, sem1)

    def fire(ci, k):
        for b in range(nb):
            pltpu.async_copy(
                table_hbm.at[idx_v.at[pl.ds((ci * nb + b) * _LPAD, L)]],
                bufs[k].at[pl.ds(b * L, L)], sems[k])

    def drain(ci, k):
        for b in range(nb):
            pltpu.make_async_copy(
                table_hbm.at[idx_v.at[pl.ds((ci * nb + b) * _LPAD, L)]],
                bufs[k].at[pl.ds(b * L, L)], sems[k]).wait()

    def consume(ci, k):
        b0 = base_b + ci * nb

        @pl.loop(0, nb)
        def _acc(b):
            def lbody(t, accs, k=k):
                out = list(accs)
                for u in range(_UNROLL):
                    r = b * L + t * _UNROLL + u
                    for j in range(_VPR):
                        out[j] = out[j] + bufs[k][r, pl.ds(j * _LANES, _LANES)]
                return tuple(out)
            accs = lax.fori_loop(
                0, L // _UNROLL, lbody,
                tuple(jnp.zeros((_LANES,), jnp.float32) for _ in range(_VPR)))
            boff = (ci * nb + b) * _LPAD
            cnt = plsc.all_reduce_population_count(
                idx_v[pl.ds(boff, _LANES)] != 0)
            for t in range(1, _LPAD // _LANES):
                cnt = cnt + plsc.all_reduce_population_count(
                    idx_v[pl.ds(boff + t * _LANES, _LANES)] != 0)
            inv = 1.0 / jnp.maximum(cnt.astype(jnp.float32), 1.0)
            for j in range(_VPR):
                stage_v[b, pl.ds(j * _LANES, _LANES)] = accs[j] * inv

        pltpu.sync_copy(stage_v, out_hbm.at[pl.ds(b0, nb)])

    fire(0, 0)

    @pl.loop(0, nchunk, step=2)
    def _pair(ci):
        fire(ci + 1, 1)
        drain(ci, 0)
        consume(ci, 0)

        @pl.when(ci + 2 < nchunk)
        def _():
            fire(ci + 2, 0)

        drain(ci + 1, 1)
        consume(ci + 1, 1)


def _make_pool(B, L):
    nb = 8
    bpw = B // _NW
    nchunk = bpw // nb
    rows = nb * L
    mesh = plsc.VectorSubcoreMesh(core_axis_name="c", subcore_axis_name="s")
    cp = pltpu.CompilerParams()
    if "needs_layout_passes" in pltpu.CompilerParams.__dataclass_fields__:
        cp = dataclasses.replace(cp, needs_layout_passes=False)
    return pl.kernel(
        functools.partial(_pool_body, nb, nchunk, L),
        out_type=jax.ShapeDtypeStruct((B, _EMB), jnp.float32),
        mesh=mesh,
        compiler_params=cp,
        scratch_types=[
            pltpu.VMEM((bpw * _LPAD,), jnp.int32),
            pltpu.VMEM((rows, _EMB), jnp.float32),
            pltpu.VMEM((rows, _EMB), jnp.float32),
            pltpu.VMEM((nb, _EMB), jnp.float32),
            pltpu.SemaphoreType.DMA,
            pltpu.SemaphoreType.DMA,
        ],
    )


def _mlp_body(p_ref, W1_ref, b1_ref, W2_ref, b2_ref, W3_ref, b3_ref, o_ref):
    dn = (((1,), (1,)), ((), ()))
    h = lax.dot_general(p_ref[...].astype(jnp.bfloat16), W1_ref[...], dn,
                        preferred_element_type=jnp.float32) + b1_ref[...]
    h = jnp.maximum(h, 0.0)
    h = lax.dot_general(h.astype(jnp.bfloat16), W2_ref[...], dn,
                        preferred_element_type=jnp.float32) + b2_ref[...]
    h = jnp.maximum(h, 0.0)
    o_ref[...] = lax.dot_general(h.astype(jnp.bfloat16), W3_ref[...], dn,
                                 preferred_element_type=jnp.float32) + b3_ref[...]


def _mlp(pooled, W1, b1, W2, b2, W3, b3):
    B = pooled.shape[0]
    H1 = W1.shape[0]
    H2 = W2.shape[0]
    OUT = W3.shape[0]
    BM = 2048
    grid = (B // BM,)
    full = lambda i: (0, 0)
    return pl.pallas_call(
        _mlp_body,
        grid=grid,
        in_specs=[
            pl.BlockSpec((BM, _EMB), lambda i: (i, 0)),
            pl.BlockSpec((H1, _EMB), full),
            pl.BlockSpec((1, H1), full),
            pl.BlockSpec((H2, H1), full),
            pl.BlockSpec((1, H2), full),
            pl.BlockSpec((OUT, H2), full),
            pl.BlockSpec((1, OUT), full),
        ],
        out_specs=pl.BlockSpec((BM, OUT), lambda i: (i, 0)),
        out_shape=jax.ShapeDtypeStruct((B, OUT), jnp.float32),
        compiler_params=pltpu.CompilerParams(
            dimension_semantics=("parallel",)),
    )(pooled, W1, b1, W2, b2, W3, b3)


def kernel(text, table, W1, b1, W2, b2, W3, b3):
    B, L = text.shape
    pool = _make_pool(B, L)
    W1b = W1.astype(jnp.bfloat16)
    W2b = W2.astype(jnp.bfloat16)
    W3b = W3.astype(jnp.bfloat16)
    tp = jnp.pad(text.astype(jnp.int32), ((0, 0), (0, _LPAD - L)))
    pooled = pool(tp.reshape(-1), table)
    return _mlp(pooled, W1b, b1.reshape(1, -1), W2b, b2.reshape(1, -1),
                W3b, b3.reshape(1, -1))


# speedup vs baseline: 1.0063x; 1.0061x over previous
import dataclasses
import functools

import jax
import jax.numpy as jnp
from jax import lax
from jax.experimental import pallas as pl
from jax.experimental.pallas import tpu as pltpu
from jax.experimental.pallas import tpu_sc as plsc

_EMB = 128
_LANES = 16
_VPR = _EMB // _LANES

_NC = 2
_NS = 16
_NW = _NC * _NS

_UNROLL = 5
_LPAD = 128


def _pool_body(nb, nchunk, L, text_hbm, table_hbm, out_hbm, idx_v, rows0,
               rows1, stage0, stage1, sem0, sem1, semw0, semw1):
    c = lax.axis_index("c")
    s = lax.axis_index("s")
    wid = s * _NC + c
    bpw = nb * nchunk
    base_b = wid * bpw
    pltpu.sync_copy(text_hbm.at[pl.ds(base_b * _LPAD, bpw * _LPAD)], idx_v)

    bufs = (rows0, rows1)
    sems = (sem0, sem1)
    stages = (stage0, stage1)
    semws = (semw0, semw1)

    def fire(ci, k):
        for b in range(nb):
            pltpu.async_copy(
                table_hbm.at[idx_v.at[pl.ds((ci * nb + b) * _LPAD, L)]],
                bufs[k].at[pl.ds(b * L, L)], sems[k])

    def drain(ci, k):
        for b in range(nb):
            pltpu.make_async_copy(
                table_hbm.at[idx_v.at[pl.ds((ci * nb + b) * _LPAD, L)]],
                bufs[k].at[pl.ds(b * L, L)], sems[k]).wait()

    def consume(ci, k):
        b0 = base_b + ci * nb
        stg = stages[k]

        @pl.when(ci >= 2)
        def _():
            pltpu.make_async_copy(stg, out_hbm.at[pl.ds(b0, nb)],
                                  semws[k]).wait()

        @pl.loop(0, nb)
        def _acc(b):
            def lbody(t, accs, k=k):
                out = list(accs)
                for u in range(_UNROLL):
                    r = b * L + t * _UNROLL + u
                    for j in range(_VPR):
                        out[j] = out[j] + bufs[k][r, pl.ds(j * _LANES, _LANES)]
                return tuple(out)
            accs = lax.fori_loop(
                0, L // _UNROLL, lbody,
                tuple(jnp.zeros((_LANES,), jnp.float32) for _ in range(_VPR)))
            boff = (ci * nb + b) * _LPAD
            cnt = plsc.all_reduce_population_count(
                idx_v[pl.ds(boff, _LANES)] != 0)
            for t in range(1, _LPAD // _LANES):
                cnt = cnt + plsc.all_reduce_population_count(
                    idx_v[pl.ds(boff + t * _LANES, _LANES)] != 0)
            inv = 1.0 / jnp.maximum(cnt.astype(jnp.float32), 1.0)
            for j in range(_VPR):
                stg[b, pl.ds(j * _LANES, _LANES)] = accs[j] * inv

        pltpu.async_copy(stg, out_hbm.at[pl.ds(b0, nb)], semws[k])

    fire(0, 0)

    @pl.loop(0, nchunk, step=2)
    def _pair(ci):
        fire(ci + 1, 1)
        drain(ci, 0)
        consume(ci, 0)

        @pl.when(ci + 2 < nchunk)
        def _():
            fire(ci + 2, 0)

        drain(ci + 1, 1)
        consume(ci + 1, 1)

    for k in (0, 1):
        pltpu.make_async_copy(stages[k], out_hbm.at[pl.ds(base_b, nb)],
                              semws[k]).wait()


def _make_pool(B, L):
    nb = 8
    bpw = B // _NW
    nchunk = bpw // nb
    rows = nb * L
    mesh = plsc.VectorSubcoreMesh(core_axis_name="c", subcore_axis_name="s")
    cp = pltpu.CompilerParams()
    if "needs_layout_passes" in pltpu.CompilerParams.__dataclass_fields__:
        cp = dataclasses.replace(cp, needs_layout_passes=False)
    return pl.kernel(
        functools.partial(_pool_body, nb, nchunk, L),
        out_type=jax.ShapeDtypeStruct((B, _EMB), jnp.float32),
        mesh=mesh,
        compiler_params=cp,
        scratch_types=[
            pltpu.VMEM((bpw * _LPAD,), jnp.int32),
            pltpu.VMEM((rows, _EMB), jnp.float32),
            pltpu.VMEM((rows, _EMB), jnp.float32),
            pltpu.VMEM((nb, _EMB), jnp.float32),
            pltpu.VMEM((nb, _EMB), jnp.float32),
            pltpu.SemaphoreType.DMA,
            pltpu.SemaphoreType.DMA,
            pltpu.SemaphoreType.DMA,
            pltpu.SemaphoreType.DMA,
        ],
    )


def _mlp_body(p_ref, W1_ref, b1_ref, W2_ref, b2_ref, W3_ref, b3_ref, o_ref):
    dn = (((1,), (1,)), ((), ()))
    h = lax.dot_general(p_ref[...].astype(jnp.bfloat16), W1_ref[...], dn,
                        preferred_element_type=jnp.float32) + b1_ref[...]
    h = jnp.maximum(h, 0.0)
    h = lax.dot_general(h.astype(jnp.bfloat16), W2_ref[...], dn,
                        preferred_element_type=jnp.float32) + b2_ref[...]
    h = jnp.maximum(h, 0.0)
    o_ref[...] = lax.dot_general(h.astype(jnp.bfloat16), W3_ref[...], dn,
                                 preferred_element_type=jnp.float32) + b3_ref[...]


def _mlp(pooled, W1, b1, W2, b2, W3, b3):
    B = pooled.shape[0]
    H1 = W1.shape[0]
    H2 = W2.shape[0]
    OUT = W3.shape[0]
    BM = 2048
    grid = (B // BM,)
    full = lambda i: (0, 0)
    return pl.pallas_call(
        _mlp_body,
        grid=grid,
        in_specs=[
            pl.BlockSpec((BM, _EMB), lambda i: (i, 0)),
            pl.BlockSpec((H1, _EMB), full),
            pl.BlockSpec((1, H1), full),
            pl.BlockSpec((H2, H1), full),
            pl.BlockSpec((1, H2), full),
            pl.BlockSpec((OUT, H2), full),
            pl.BlockSpec((1, OUT), full),
        ],
        out_specs=pl.BlockSpec((BM, OUT), lambda i: (i, 0)),
        out_shape=jax.ShapeDtypeStruct((B, OUT), jnp.float32),
    )(pooled, W1, b1, W2, b2, W3, b3)


def kernel(text, table, W1, b1, W2, b2, W3, b3):
    B, L = text.shape
    pool = _make_pool(B, L)
    W1b = W1.astype(jnp.bfloat16)
    W2b = W2.astype(jnp.bfloat16)
    W3b = W3.astype(jnp.bfloat16)
    tp = jnp.pad(text.astype(jnp.int32), ((0, 0), (0, _LPAD - L)))
    pooled = pool(tp.reshape(-1), table)
    return _mlp(pooled, W1b, b1.reshape(1, -1), W2b, b2.reshape(1, -1),
                W3b, b3.reshape(1, -1))
